# baseline (speedup 1.0000x reference)
import functools

import jax
import jax.numpy as jnp
import numpy as np
from jax import lax
from jax.experimental import pallas as pl
from jax.experimental.pallas import tpu as pltpu
from jax.experimental.pallas import tpu_sc as plsc

N_ATOMS = 10000
N_EDGES = 320000
F = 128
R = 16
L = 16
G = F // 32
FP = F // 2
NC = 2
NS = 16
NW = NC * NS
EDGES_PER_W = N_EDGES // NW
CHUNK = 40
N_CHUNKS = EDGES_PER_W // CHUNK
N_ATOMS_PAD = 10240
ROWS_PER_TILE = N_ATOMS_PAD // NS

_HI_MASK = np.int32(-65536)

_PERM2 = np.concatenate([
    np.concatenate([np.arange(g * 32, g * 32 + 32, 2),
                    np.arange(g * 32 + 1, g * 32 + 32, 2)])
    for g in range(G)
])


_BE = 8000


def _filter_body(rbf_ref, w1_ref, b1_ref, w2_ref, b2_ref, out_ref):
    x = jnp.dot(rbf_ref[...].astype(jnp.bfloat16),
                w1_ref[...].astype(jnp.bfloat16),
                preferred_element_type=jnp.float32)
    x = x + b1_ref[...]
    x = x * jax.nn.sigmoid(x)
    y = jnp.dot(x.astype(jnp.bfloat16),
                w2_ref[...].astype(jnp.bfloat16),
                preferred_element_type=jnp.float32) + b2_ref[...]
    out_ref[...] = y.astype(jnp.bfloat16)


def _filter_mlp(rbf, w1, b1, w2, b2):
    grid = (N_EDGES // _BE,)
    return pl.pallas_call(
        _filter_body,
        grid=grid,
        in_specs=[
            pl.BlockSpec((_BE, R), lambda i: (i, 0)),
            pl.BlockSpec((R, F), lambda i: (0, 0)),
            pl.BlockSpec((1, F), lambda i: (0, 0)),
            pl.BlockSpec((F, F), lambda i: (0, 0)),
            pl.BlockSpec((1, F), lambda i: (0, 0)),
        ],
        out_specs=pl.BlockSpec((_BE, F), lambda i: (i, 0)),
        out_shape=jax.ShapeDtypeStruct((N_EDGES, F), jnp.bfloat16),
    )(rbf, w1, b1, w2, b2)


def _pack_i32(x_bf16):
    rows = x_bf16.shape[0]
    return lax.bitcast_convert_type(
        x_bf16.reshape(rows, FP, 2), jnp.int32)


_SC_MESH = plsc.VectorSubcoreMesh(core_axis_name="c", subcore_axis_name="s")


@functools.partial(
    pl.kernel,
    out_type=jax.ShapeDtypeStruct((NC, N_ATOMS_PAD, F), jnp.float32),
    mesh=_SC_MESH,
    scratch_types=[
        pltpu.VMEM((4, CHUNK), jnp.int32),
        pltpu.VMEM((4, CHUNK), jnp.int32),
        pltpu.VMEM((2, CHUNK, F), jnp.float32),
        pltpu.VMEM((2, CHUNK, FP), jnp.int32),
        pltpu.VMEM((2, CHUNK, F), jnp.float32),
        pltpu.VMEM_SHARED((N_ATOMS_PAD, F), jnp.float32),
        pltpu.SemaphoreType.DMA,
        pltpu.SemaphoreType.DMA,
        pltpu.SemaphoreType.DMA,
        pltpu.SemaphoreType.DMA,
        pltpu.SemaphoreType.DMA,
        pltpu.SemaphoreType.DMA,
        pltpu.SemaphoreType.DMA,
        pltpu.SemaphoreType.DMA,
        pltpu.SemaphoreType.DMA,
        pltpu.SemaphoreType.DMA,
    ],
)
def _sc_aggregate(h_hbm, w_hbm, idxi3_hbm, idxj3_hbm, zeros_hbm, out_hbm,
                  idxj_v, idxi_v, rows_v, wrows_v, msg_v, agg_sh,
                  gsem0, gsem1, wsem0, wsem1, ssem0, ssem1,
                  isem0, isem1, isem2, isem3):
    gsems = (gsem0, gsem1)
    wsems = (wsem0, wsem1)
    ssems = (ssem0, ssem1)
    isems = (isem0, isem1, isem2, isem3)
    c = lax.axis_index("c")
    s = lax.axis_index("s")
    wid = c * NS + s

    row0 = s * ROWS_PER_TILE
    pltpu.sync_copy(
        zeros_hbm.at[pl.ds(row0, ROWS_PER_TILE)],
        agg_sh.at[pl.ds(row0, ROWS_PER_TILE)],
    )
    plsc.subcore_barrier()

    base_edge = wid * EDGES_PER_W

    def idxj_fetch_desc(ci, q):
        return pltpu.make_async_copy(idxj3_hbm.at[wid, ci], idxj_v.at[q],
                                     isems[q])

    def idxi_fetch_desc(ci, q):
        return pltpu.make_async_copy(idxi3_hbm.at[wid, ci], idxi_v.at[q],
                                     isems[q])

    def gather_desc(q, b):
        return pltpu.make_async_copy(
            h_hbm.at[idxj_v.at[q]], rows_v.at[b], gsems[b])

    def wrow_desc(ci, b):
        return pltpu.make_async_copy(
            w_hbm.at[pl.ds(base_edge + ci * CHUNK, CHUNK)],
            wrows_v.at[b], wsems[b])

    def scat_desc(q, b):
        return pltpu.make_async_copy(
            msg_v.at[b], agg_sh.at[idxi_v.at[q]], ssems[b])

    def start_fetch(ci, q, b):
        gather_desc(q, b).start()
        wrow_desc(ci, b).start()

    for ci0 in range(2):
        pltpu.sync_copy(idxj3_hbm.at[wid, ci0], idxj_v.at[ci0])
        pltpu.sync_copy(idxi3_hbm.at[wid, ci0], idxi_v.at[ci0])
        start_fetch(ci0, ci0, ci0)

    def process(ci, q, b, guard_drain, do_prefetch):
        gather_desc(q, b).wait()
        wrow_desc(ci, b).wait()

        def _drain_prev_scatter():
            scat_desc((q + 2) % 4, b).wait()

        if guard_drain:
            pl.when(ci >= 2)(_drain_prev_scatter)
        else:
            _drain_prev_scatter()

        if do_prefetch:
            idxj_fetch_desc(ci + 2, (q + 2) % 4).start()
            idxi_fetch_desc(ci + 2, (q + 2) % 4).start()

        @plsc.parallel_loop(0, CHUNK, 1, unroll=2)
        def _mul(e):
            for g in range(G):
                wv = wrows_v[b, e, pl.ds(g * L, L)]
                wlo = lax.bitcast_convert_type(wv << 16, jnp.float32)
                whi = lax.bitcast_convert_type(wv & _HI_MASK, jnp.float32)
                msg_v[b, e, pl.ds(g * 32, L)] = (
                    rows_v[b, e, pl.ds(g * 32, L)] * wlo)
                msg_v[b, e, pl.ds(g * 32 + L, L)] = (
                    rows_v[b, e, pl.ds(g * 32 + L, L)] * whi)

        pltpu.async_copy(
            msg_v.at[b], agg_sh.at[idxi_v.at[q]], ssems[b], add=True)

        if do_prefetch:
            idxj_fetch_desc(ci + 2, (q + 2) % 4).wait()
            idxi_fetch_desc(ci + 2, (q + 2) % 4).wait()
            start_fetch(ci + 2, (q + 2) % 4, b)

    def quad_body(g, carry):
        for q in range(4):
            ci = 4 * g + q
            process(ci, q, b=q % 2, guard_drain=(q < 2), do_prefetch=True)
        return carry

    lax.fori_loop(0, N_CHUNKS // 4, quad_body, 0)
    process(N_CHUNKS - 2, 0, 0, guard_drain=False, do_prefetch=False)
    process(N_CHUNKS - 1, 1, 1, guard_drain=False, do_prefetch=False)
    scat_desc(0, 0).wait()
    scat_desc(1, 1).wait()
    plsc.subcore_barrier()

    pltpu.sync_copy(
        agg_sh.at[pl.ds(row0, ROWS_PER_TILE)],
        out_hbm.at[c, pl.ds(row0, ROWS_PER_TILE)],
    )


_BA = 2000


def _update_body(h_ref, p_ref, w3_ref, b3_ref, w4_ref, b4_ref, out_ref):
    agg = p_ref[0] + p_ref[1]
    x = jnp.dot(agg, w3_ref[...], preferred_element_type=jnp.float32) + b3_ref[...]
    x = x * jax.nn.sigmoid(x)
    out_ref[...] = (
        h_ref[...]
        + jnp.dot(x, w4_ref[...], preferred_element_type=jnp.float32)
        + b4_ref[...]
    )


def _update_mlp(h, partials, w3, b3, w4, b4):
    grid = (N_ATOMS // _BA,)
    return pl.pallas_call(
        _update_body,
        grid=grid,
        in_specs=[
            pl.BlockSpec((_BA, F), lambda i: (i, 0)),
            pl.BlockSpec((NC, _BA, F), lambda i: (0, i, 0)),
            pl.BlockSpec((F, F), lambda i: (0, 0)),
            pl.BlockSpec((1, F), lambda i: (0, 0)),
            pl.BlockSpec((F, F), lambda i: (0, 0)),
            pl.BlockSpec((1, F), lambda i: (0, 0)),
        ],
        out_specs=pl.BlockSpec((_BA, F), lambda i: (i, 0)),
        out_shape=jax.ShapeDtypeStruct((N_ATOMS, F), jnp.float32),
    )(h, partials, w3, b3, w4, b4)


def kernel(h, rbf_ij, idx_i, idx_j, W1, b1, W2, b2, W3, b3, W4, b4):
    idx_i3 = idx_i.astype(jnp.int32).reshape(NW, N_CHUNKS, CHUNK)
    idx_j3 = idx_j.astype(jnp.int32).reshape(NW, N_CHUNKS, CHUNK)
    h_g = h[:, jnp.asarray(_PERM2)]
    w_pk = _pack_i32(_filter_mlp(rbf_ij, W1, b1.reshape(1, F),
                                 W2, b2.reshape(1, F)))
    zeros = jnp.zeros((N_ATOMS_PAD, F), jnp.float32)
    partials = _sc_aggregate(h_g, w_pk, idx_i3, idx_j3, zeros)
    w3p = W3[jnp.asarray(_PERM2), :]
    return _update_mlp(h, partials, w3p, b3.reshape(1, F), W4, b4.reshape(1, F))


# speedup vs baseline: 2.9145x; 2.9145x over previous
import functools

import jax
import jax.numpy as jnp
from jax import lax
from jax.experimental import pallas as pl
from jax.experimental.pallas import tpu as pltpu
from jax.experimental.pallas import tpu_sc as plsc

N_ATOMS = 10000
N_EDGES = 320000
F = 128
R = 16
L = 16
NC = 2
NS = 16
NW = NC * NS
EDGES_PER_W = N_EDGES // NW
CHUNK = 40
N_CHUNKS = EDGES_PER_W // CHUNK
N_ATOMS_PAD = 10240
ROWS_PER_TILE = N_ATOMS_PAD // NS


_BE = 8000


def _filter_body(rbf_ref, w1_ref, b1_ref, w2_ref, b2_ref, out_ref):
    x = jnp.dot(rbf_ref[...].astype(jnp.bfloat16),
                w1_ref[...].astype(jnp.bfloat16),
                preferred_element_type=jnp.float32)
    x = x + b1_ref[...]
    x = x * jax.nn.sigmoid(x)
    y = jnp.dot(x.astype(jnp.bfloat16),
                w2_ref[...].astype(jnp.bfloat16),
                preferred_element_type=jnp.float32) + b2_ref[...]
    out_ref[...] = y


def _filter_mlp(rbf, w1, b1, w2, b2):
    grid = (N_EDGES // _BE,)
    return pl.pallas_call(
        _filter_body,
        grid=grid,
        in_specs=[
            pl.BlockSpec((_BE, R), lambda i: (i, 0)),
            pl.BlockSpec((R, F), lambda i: (0, 0)),
            pl.BlockSpec((1, F), lambda i: (0, 0)),
            pl.BlockSpec((F, F), lambda i: (0, 0)),
            pl.BlockSpec((1, F), lambda i: (0, 0)),
        ],
        out_specs=pl.BlockSpec((_BE, F), lambda i: (i, 0)),
        out_shape=jax.ShapeDtypeStruct((N_EDGES, F), jnp.float32),
    )(rbf, w1, b1, w2, b2)


_SC_MESH = plsc.VectorSubcoreMesh(core_axis_name="c", subcore_axis_name="s")


@functools.partial(
    pl.kernel,
    out_type=jax.ShapeDtypeStruct((NC, N_ATOMS_PAD, F), jnp.float32),
    mesh=_SC_MESH,
    scratch_types=[
        pltpu.VMEM((4, CHUNK), jnp.int32),
        pltpu.VMEM((4, CHUNK), jnp.int32),
        pltpu.VMEM((2, CHUNK, F), jnp.float32),
        pltpu.VMEM((2, CHUNK, F), jnp.float32),
        pltpu.VMEM((2, CHUNK, F), jnp.float32),
        pltpu.VMEM_SHARED((N_ATOMS_PAD, F), jnp.float32),
        pltpu.SemaphoreType.DMA,
        pltpu.SemaphoreType.DMA,
        pltpu.SemaphoreType.DMA,
        pltpu.SemaphoreType.DMA,
        pltpu.SemaphoreType.DMA,
        pltpu.SemaphoreType.DMA,
        pltpu.SemaphoreType.DMA,
        pltpu.SemaphoreType.DMA,
        pltpu.SemaphoreType.DMA,
        pltpu.SemaphoreType.DMA,
    ],
)
def _sc_aggregate(h_hbm, w_hbm, idxi3_hbm, idxj3_hbm, zeros_hbm, out_hbm,
                  idxj_v, idxi_v, rows_v, wrows_v, msg_v, agg_sh,
                  gsem0, gsem1, wsem0, wsem1, ssem0, ssem1,
                  isem0, isem1, isem2, isem3):
    gsems = (gsem0, gsem1)
    wsems = (wsem0, wsem1)
    ssems = (ssem0, ssem1)
    isems = (isem0, isem1, isem2, isem3)
    c = lax.axis_index("c")
    s = lax.axis_index("s")
    wid = c * NS + s

    row0 = s * ROWS_PER_TILE
    pltpu.sync_copy(
        zeros_hbm.at[pl.ds(row0, ROWS_PER_TILE)],
        agg_sh.at[pl.ds(row0, ROWS_PER_TILE)],
    )
    plsc.subcore_barrier()

    base_edge = wid * EDGES_PER_W

    def idxj_fetch_desc(ci, q):
        return pltpu.make_async_copy(idxj3_hbm.at[wid, ci], idxj_v.at[q],
                                     isems[q])

    def idxi_fetch_desc(ci, q):
        return pltpu.make_async_copy(idxi3_hbm.at[wid, ci], idxi_v.at[q],
                                     isems[q])

    def gather_desc(q, b):
        return pltpu.make_async_copy(
            h_hbm.at[idxj_v.at[q]], rows_v.at[b], gsems[b])

    def wrow_desc(ci, b):
        return pltpu.make_async_copy(
            w_hbm.at[pl.ds(base_edge + ci * CHUNK, CHUNK)],
            wrows_v.at[b], wsems[b])

    def scat_desc(q, b):
        return pltpu.make_async_copy(
            msg_v.at[b], agg_sh.at[idxi_v.at[q]], ssems[b])

    def start_fetch(ci, q, b):
        gather_desc(q, b).start()
        wrow_desc(ci, b).start()

    for ci0 in range(2):
        pltpu.sync_copy(idxj3_hbm.at[wid, ci0], idxj_v.at[ci0])
        pltpu.sync_copy(idxi3_hbm.at[wid, ci0], idxi_v.at[ci0])
        start_fetch(ci0, ci0, ci0)

    def process(ci, q, b, guard_drain, do_prefetch):
        gather_desc(q, b).wait()
        wrow_desc(ci, b).wait()

        def _drain_prev_scatter():
            scat_desc((q + 2) % 4, b).wait()

        if guard_drain:
            pl.when(ci >= 2)(_drain_prev_scatter)
        else:
            _drain_prev_scatter()

        if do_prefetch:
            idxj_fetch_desc(ci + 2, (q + 2) % 4).start()
            idxi_fetch_desc(ci + 2, (q + 2) % 4).start()

        @plsc.parallel_loop(0, CHUNK, 1, unroll=2)
        def _mul(e):
            for g in range(F // L):
                msg_v[b, e, pl.ds(g * L, L)] = (
                    rows_v[b, e, pl.ds(g * L, L)]
                    * wrows_v[b, e, pl.ds(g * L, L)])

        pltpu.async_copy(
            msg_v.at[b], agg_sh.at[idxi_v.at[q]], ssems[b], add=True)

        if do_prefetch:
            idxj_fetch_desc(ci + 2, (q + 2) % 4).wait()
            idxi_fetch_desc(ci + 2, (q + 2) % 4).wait()
            start_fetch(ci + 2, (q + 2) % 4, b)

    def quad_body(g, carry):
        for q in range(4):
            ci = 4 * g + q
            process(ci, q, b=q % 2, guard_drain=(q < 2), do_prefetch=True)
        return carry

    lax.fori_loop(0, N_CHUNKS // 4, quad_body, 0)
    process(N_CHUNKS - 2, 0, 0, guard_drain=False, do_prefetch=False)
    process(N_CHUNKS - 1, 1, 1, guard_drain=False, do_prefetch=False)
    scat_desc(0, 0).wait()
    scat_desc(1, 1).wait()
    plsc.subcore_barrier()

    pltpu.sync_copy(
        agg_sh.at[pl.ds(row0, ROWS_PER_TILE)],
        out_hbm.at[c, pl.ds(row0, ROWS_PER_TILE)],
    )


_BA = 2000


def _update_body(h_ref, p_ref, w3_ref, b3_ref, w4_ref, b4_ref, out_ref):
    agg = p_ref[0] + p_ref[1]
    x = jnp.dot(agg, w3_ref[...], preferred_element_type=jnp.float32) + b3_ref[...]
    x = x * jax.nn.sigmoid(x)
    out_ref[...] = (
        h_ref[...]
        + jnp.dot(x, w4_ref[...], preferred_element_type=jnp.float32)
        + b4_ref[...]
    )


def _update_mlp(h, partials, w3, b3, w4, b4):
    grid = (N_ATOMS // _BA,)
    return pl.pallas_call(
        _update_body,
        grid=grid,
        in_specs=[
            pl.BlockSpec((_BA, F), lambda i: (i, 0)),
            pl.BlockSpec((NC, _BA, F), lambda i: (0, i, 0)),
            pl.BlockSpec((F, F), lambda i: (0, 0)),
            pl.BlockSpec((1, F), lambda i: (0, 0)),
            pl.BlockSpec((F, F), lambda i: (0, 0)),
            pl.BlockSpec((1, F), lambda i: (0, 0)),
        ],
        out_specs=pl.BlockSpec((_BA, F), lambda i: (i, 0)),
        out_shape=jax.ShapeDtypeStruct((N_ATOMS, F), jnp.float32),
    )(h, partials, w3, b3, w4, b4)


def kernel(h, rbf_ij, idx_i, idx_j, W1, b1, W2, b2, W3, b3, W4, b4):
    idx_i3 = idx_i.astype(jnp.int32).reshape(NW, N_CHUNKS, CHUNK)
    idx_j3 = idx_j.astype(jnp.int32).reshape(NW, N_CHUNKS, CHUNK)
    w_all = _filter_mlp(rbf_ij, W1, b1.reshape(1, F), W2, b2.reshape(1, F))
    zeros = jnp.zeros((N_ATOMS_PAD, F), jnp.float32)
    partials = _sc_aggregate(h, w_all, idx_i3, idx_j3, zeros)
    return _update_mlp(h, partials, W3, b3.reshape(1, F), W4, b4.reshape(1, F))


# speedup vs baseline: 3.9436x; 1.3531x over previous
import functools

import jax
import jax.numpy as jnp
from jax import lax
from jax.experimental import pallas as pl
from jax.experimental.pallas import tpu as pltpu
from jax.experimental.pallas import tpu_sc as plsc

N_ATOMS = 10000
N_EDGES = 320000
F = 128
R = 16
L = 16
NC = 2
NS = 16
NW = NC * NS
EDGES_PER_W = N_EDGES // NW
CHUNK = 40
N_CHUNKS = EDGES_PER_W // CHUNK
N_ATOMS_PAD = 10240
ROWS_PER_TILE = N_ATOMS_PAD // NS


_BE = 12800


def _filter_body(rbf_ref, w1_ref, b1_ref, w2_ref, b2_ref, out_ref):
    rbf_blk = rbf_ref[...].T
    x = jnp.dot(rbf_blk.astype(jnp.bfloat16),
                w1_ref[...].astype(jnp.bfloat16),
                preferred_element_type=jnp.float32)
    x = x + b1_ref[...]
    x = x * jax.nn.sigmoid(x)
    y = jnp.dot(x.astype(jnp.bfloat16),
                w2_ref[...].astype(jnp.bfloat16),
                preferred_element_type=jnp.float32) + b2_ref[...]
    out_ref[...] = y


def _filter_mlp(rbf_t, w1, b1, w2, b2):
    grid = (N_EDGES // _BE,)
    return pl.pallas_call(
        _filter_body,
        grid=grid,
        in_specs=[
            pl.BlockSpec((R, _BE), lambda i: (0, i)),
            pl.BlockSpec((R, F), lambda i: (0, 0)),
            pl.BlockSpec((1, F), lambda i: (0, 0)),
            pl.BlockSpec((F, F), lambda i: (0, 0)),
            pl.BlockSpec((1, F), lambda i: (0, 0)),
        ],
        out_specs=pl.BlockSpec((_BE, F), lambda i: (i, 0)),
        out_shape=jax.ShapeDtypeStruct((N_EDGES, F), jnp.float32),
    )(rbf_t, w1, b1, w2, b2)


_SC_MESH = plsc.VectorSubcoreMesh(core_axis_name="c", subcore_axis_name="s")


@functools.partial(
    pl.kernel,
    out_type=jax.ShapeDtypeStruct((NC, N_ATOMS_PAD, F), jnp.float32),
    mesh=_SC_MESH,
    scratch_types=[
        pltpu.VMEM((4, CHUNK), jnp.int32),
        pltpu.VMEM((4, CHUNK), jnp.int32),
        pltpu.VMEM((2, CHUNK, F), jnp.float32),
        pltpu.VMEM((2, CHUNK, F), jnp.float32),
        pltpu.VMEM((2, CHUNK, F), jnp.float32),
        pltpu.VMEM_SHARED((N_ATOMS_PAD, F), jnp.float32),
        pltpu.SemaphoreType.DMA,
        pltpu.SemaphoreType.DMA,
        pltpu.SemaphoreType.DMA,
        pltpu.SemaphoreType.DMA,
        pltpu.SemaphoreType.DMA,
        pltpu.SemaphoreType.DMA,
        pltpu.SemaphoreType.DMA,
        pltpu.SemaphoreType.DMA,
        pltpu.SemaphoreType.DMA,
        pltpu.SemaphoreType.DMA,
    ],
)
def _sc_aggregate(h_hbm, w_hbm, idxi3_hbm, idxj3_hbm, zeros_hbm, out_hbm,
                  idxj_v, idxi_v, rows_v, wrows_v, msg_v, agg_sh,
                  gsem0, gsem1, wsem0, wsem1, ssem0, ssem1,
                  isem0, isem1, isem2, isem3):
    gsems = (gsem0, gsem1)
    wsems = (wsem0, wsem1)
    ssems = (ssem0, ssem1)
    isems = (isem0, isem1, isem2, isem3)
    c = lax.axis_index("c")
    s = lax.axis_index("s")
    wid = c * NS + s

    row0 = s * ROWS_PER_TILE
    pltpu.sync_copy(
        zeros_hbm.at[pl.ds(row0, ROWS_PER_TILE)],
        agg_sh.at[pl.ds(row0, ROWS_PER_TILE)],
    )
    plsc.subcore_barrier()

    base_edge = wid * EDGES_PER_W

    def idxj_fetch_desc(ci, q):
        return pltpu.make_async_copy(idxj3_hbm.at[wid, ci], idxj_v.at[q],
                                     isems[q])

    def idxi_fetch_desc(ci, q):
        return pltpu.make_async_copy(idxi3_hbm.at[wid, ci], idxi_v.at[q],
                                     isems[q])

    def gather_desc(q, b):
        return pltpu.make_async_copy(
            h_hbm.at[idxj_v.at[q]], rows_v.at[b], gsems[b])

    def wrow_desc(ci, b):
        return pltpu.make_async_copy(
            w_hbm.at[pl.ds(base_edge + ci * CHUNK, CHUNK)],
            wrows_v.at[b], wsems[b])

    def scat_desc(q, b):
        return pltpu.make_async_copy(
            msg_v.at[b], agg_sh.at[idxi_v.at[q]], ssems[b])

    def start_fetch(ci, q, b):
        gather_desc(q, b).start()
        wrow_desc(ci, b).start()

    for ci0 in range(2):
        pltpu.sync_copy(idxj3_hbm.at[wid, ci0], idxj_v.at[ci0])
        pltpu.sync_copy(idxi3_hbm.at[wid, ci0], idxi_v.at[ci0])
        start_fetch(ci0, ci0, ci0)

    def process(ci, q, b, guard_drain, do_prefetch):
        gather_desc(q, b).wait()
        wrow_desc(ci, b).wait()

        def _drain_prev_scatter():
            scat_desc((q + 2) % 4, b).wait()

        if guard_drain:
            pl.when(ci >= 2)(_drain_prev_scatter)
        else:
            _drain_prev_scatter()

        if do_prefetch:
            idxj_fetch_desc(ci + 2, (q + 2) % 4).start()
            idxi_fetch_desc(ci + 2, (q + 2) % 4).start()

        @plsc.parallel_loop(0, CHUNK, 1, unroll=2)
        def _mul(e):
            for g in range(F // L):
                msg_v[b, e, pl.ds(g * L, L)] = (
                    rows_v[b, e, pl.ds(g * L, L)]
                    * wrows_v[b, e, pl.ds(g * L, L)])

        pltpu.async_copy(
            msg_v.at[b], agg_sh.at[idxi_v.at[q]], ssems[b], add=True)

        if do_prefetch:
            idxj_fetch_desc(ci + 2, (q + 2) % 4).wait()
            idxi_fetch_desc(ci + 2, (q + 2) % 4).wait()
            start_fetch(ci + 2, (q + 2) % 4, b)

    def quad_body(g, carry):
        for q in range(4):
            ci = 4 * g + q
            process(ci, q, b=q % 2, guard_drain=(q < 2), do_prefetch=True)
        return carry

    lax.fori_loop(0, N_CHUNKS // 4, quad_body, 0)
    process(N_CHUNKS - 2, 0, 0, guard_drain=False, do_prefetch=False)
    process(N_CHUNKS - 1, 1, 1, guard_drain=False, do_prefetch=False)
    scat_desc(0, 0).wait()
    scat_desc(1, 1).wait()
    plsc.subcore_barrier()

    pltpu.sync_copy(
        agg_sh.at[pl.ds(row0, ROWS_PER_TILE)],
        out_hbm.at[c, pl.ds(row0, ROWS_PER_TILE)],
    )


_BA = 2000


def _update_body(h_ref, p_ref, w3_ref, b3_ref, w4_ref, b4_ref, out_ref):
    agg = p_ref[0] + p_ref[1]
    x = jnp.dot(agg, w3_ref[...], preferred_element_type=jnp.float32) + b3_ref[...]
    x = x * jax.nn.sigmoid(x)
    out_ref[...] = (
        h_ref[...]
        + jnp.dot(x, w4_ref[...], preferred_element_type=jnp.float32)
        + b4_ref[...]
    )


def _update_mlp(h, partials, w3, b3, w4, b4):
    grid = (N_ATOMS // _BA,)
    return pl.pallas_call(
        _update_body,
        grid=grid,
        in_specs=[
            pl.BlockSpec((_BA, F), lambda i: (i, 0)),
            pl.BlockSpec((NC, _BA, F), lambda i: (0, i, 0)),
            pl.BlockSpec((F, F), lambda i: (0, 0)),
            pl.BlockSpec((1, F), lambda i: (0, 0)),
            pl.BlockSpec((F, F), lambda i: (0, 0)),
            pl.BlockSpec((1, F), lambda i: (0, 0)),
        ],
        out_specs=pl.BlockSpec((_BA, F), lambda i: (i, 0)),
        out_shape=jax.ShapeDtypeStruct((N_ATOMS, F), jnp.float32),
    )(h, partials, w3, b3, w4, b4)


def kernel(h, rbf_ij, idx_i, idx_j, W1, b1, W2, b2, W3, b3, W4, b4):
    idx_i3 = idx_i.astype(jnp.int32).reshape(NW, N_CHUNKS, CHUNK)
    idx_j3 = idx_j.astype(jnp.int32).reshape(NW, N_CHUNKS, CHUNK)
    w_all = _filter_mlp(rbf_ij.T, W1, b1.reshape(1, F), W2, b2.reshape(1, F))
    zeros = jnp.zeros((N_ATOMS_PAD, F), jnp.float32)
    partials = _sc_aggregate(h, w_all, idx_i3, idx_j3, zeros)
    return _update_mlp(h, partials, W3, b3.reshape(1, F), W4, b4.reshape(1, F))
